# baseline (speedup 1.0000x reference)
import functools

import jax
import jax.numpy as jnp
from jax import lax
from jax.experimental import pallas as pl
from jax.experimental.pallas import tpu as pltpu
from jax.experimental.pallas import tpu_sc as plsc

N = 10000
E = 320000
D = 128
NC = 2
NS = 16
NW = NC * NS
NP = 10240
CH = 64
EPW = 10240
EP = NW * EPW
NCHUNK = EPW // CH
NBUF = 4
RPT = NP // NS

_mesh = plsc.VectorSubcoreMesh(core_axis_name="c", subcore_axis_name="s")


def _agg_body(with_deg, *refs):
    if with_deg:
        (table_hbm, src_hbm, dst_hbm, zrows_hbm, zcol_hbm, ones_hbm,
         psum_hbm, pdeg_hbm,
         acc_sh, deg_sh, sidx_v, didx_v, rows_v, ones_v,
         g0, g1, g2, g3, s0, s1, s2, s3) = refs
    else:
        (table_hbm, src_hbm, dst_hbm, zrows_hbm,
         psum_hbm,
         acc_sh, sidx_v, didx_v, rows_v,
         g0, g1, g2, g3, s0, s1, s2, s3) = refs
    gsems = (g0, g1, g2, g3)
    ssems = (s0, s1, s2, s3)

    cid = lax.axis_index("c")
    sid = lax.axis_index("s")
    wid = sid * NC + cid
    r0 = sid * RPT

    pltpu.sync_copy(zrows_hbm, acc_sh.at[pl.ds(r0, RPT)])
    if with_deg:
        pltpu.sync_copy(zcol_hbm, deg_sh.at[pl.ds(r0, RPT)])
        pltpu.sync_copy(ones_hbm, ones_v)
    plsc.subcore_barrier()

    ebase = wid * EPW

    def load_and_gather(c, b):
        base = pl.multiple_of(ebase + c * CH, 8)
        pltpu.sync_copy(src_hbm.at[pl.ds(base, CH)], sidx_v.at[b])
        pltpu.sync_copy(dst_hbm.at[pl.ds(base, CH)], didx_v.at[b])
        pltpu.async_copy(table_hbm.at[sidx_v.at[b]], rows_v.at[b], gsems[b])

    for b in range(NBUF):
        load_and_gather(b, b)

    def group(g, carry):
        for b in range(NBUF):
            pltpu.make_async_copy(table_hbm.at[sidx_v.at[b]],
                                  rows_v.at[b], gsems[b]).wait()
            pltpu.async_copy(rows_v.at[b], acc_sh.at[didx_v.at[b]],
                             ssems[b], add=True)
            if with_deg:
                pltpu.async_copy(ones_v, deg_sh.at[didx_v.at[b]],
                                 ssems[b], add=True)
        for b in range(NBUF):
            c2 = NBUF * g + b + NBUF
            pltpu.make_async_copy(rows_v.at[b], acc_sh.at[didx_v.at[b]],
                                  ssems[b]).wait()
            if with_deg:
                pltpu.make_async_copy(ones_v, deg_sh.at[didx_v.at[b]],
                                      ssems[b]).wait()

            @pl.when(c2 < NCHUNK)
            def _():
                load_and_gather(c2, b)
        return carry

    lax.fori_loop(0, NCHUNK // NBUF, group, 0)

    plsc.subcore_barrier()
    pltpu.sync_copy(acc_sh.at[pl.ds(r0, RPT)], psum_hbm.at[cid, pl.ds(r0, RPT)])
    if with_deg:
        pltpu.sync_copy(deg_sh.at[pl.ds(r0, RPT)],
                        pdeg_hbm.at[cid, pl.ds(r0, RPT)])


def _sc_agg_deg(table, src, dst, zrows, zcol, ones):
    f = pl.kernel(
        functools.partial(_agg_body, True),
        mesh=_mesh,
        out_type=[
            jax.ShapeDtypeStruct((NC, NP, D), jnp.float32),
            jax.ShapeDtypeStruct((NC, NP), jnp.float32),
        ],
        scratch_types=[
            pltpu.VMEM_SHARED((NP, D), jnp.float32),
            pltpu.VMEM_SHARED((NP,), jnp.float32),
            pltpu.VMEM((NBUF, CH), jnp.int32),
            pltpu.VMEM((NBUF, CH), jnp.int32),
            pltpu.VMEM((NBUF, CH, D), jnp.float32),
            pltpu.VMEM((CH,), jnp.float32),
        ] + [pltpu.SemaphoreType.DMA] * (2 * NBUF),
    )
    return f(table, src, dst, zrows, zcol, ones)


def _sc_agg(table, src, dst, zrows):
    f = pl.kernel(
        functools.partial(_agg_body, False),
        mesh=_mesh,
        out_type=jax.ShapeDtypeStruct((NC, NP, D), jnp.float32),
        scratch_types=[
            pltpu.VMEM_SHARED((NP, D), jnp.float32),
            pltpu.VMEM((NBUF, CH), jnp.int32),
            pltpu.VMEM((NBUF, CH), jnp.int32),
            pltpu.VMEM((NBUF, CH, D), jnp.float32),
        ] + [pltpu.SemaphoreType.DMA] * (2 * NBUF),
    )
    return f(table, src, dst, zrows)


_DOT = functools.partial(
    lax.dot_general,
    dimension_numbers=(((1,), (0,)), ((), ())),
    preferred_element_type=jnp.float32,
    precision=lax.Precision.HIGHEST,
)

_R = 1024


def _dense1_body(ps_ref, dg_ref, x_ref, wl_ref, wr_ref, b_ref, o_ref):
    s = ps_ref[0] + ps_ref[1]
    dg = dg_ref[0] + dg_ref[1]
    agg = s / jnp.maximum(dg, 1.0)
    h = _DOT(agg, wl_ref[...]) + b_ref[...] + _DOT(x_ref[...], wr_ref[...])
    o_ref[...] = jnp.maximum(h, 0.0)


def _dense1(psum, pdeg3, x, wl, wr, b):
    grid = (NP // _R,)
    return pl.pallas_call(
        _dense1_body,
        grid=grid,
        in_specs=[
            pl.BlockSpec((NC, _R, D), lambda i: (0, i, 0)),
            pl.BlockSpec((NC, _R, 1), lambda i: (0, i, 0)),
            pl.BlockSpec((_R, D), lambda i: (i, 0)),
            pl.BlockSpec((D, D), lambda i: (0, 0)),
            pl.BlockSpec((D, D), lambda i: (0, 0)),
            pl.BlockSpec((1, D), lambda i: (0, 0)),
        ],
        out_specs=pl.BlockSpec((_R, D), lambda i: (i, 0)),
        out_shape=jax.ShapeDtypeStruct((NP, D), jnp.float32),
    )(psum, pdeg3, x, wl, wr, b)


def _dense2_body(ps_ref, dg_ref, h1_ref, wl_ref, wr_ref, b_ref,
                 wa_ref, wb_ref, bro_ref, o_ref):
    s = ps_ref[0] + ps_ref[1]
    dg = dg_ref[0] + dg_ref[1]
    agg = s / jnp.maximum(dg, 1.0)
    h1 = h1_ref[...]
    h2 = jnp.maximum(_DOT(agg, wl_ref[...]) + b_ref[...]
                     + _DOT(h1, wr_ref[...]), 0.0)
    o_ref[...] = _DOT(h1, wa_ref[...]) + _DOT(h2, wb_ref[...]) + bro_ref[...]


def _dense2(psum, pdeg3, h1, wl, wr, b, wa, wb, bro):
    grid = (NP // _R,)
    return pl.pallas_call(
        _dense2_body,
        grid=grid,
        in_specs=[
            pl.BlockSpec((NC, _R, D), lambda i: (0, i, 0)),
            pl.BlockSpec((NC, _R, 1), lambda i: (0, i, 0)),
            pl.BlockSpec((_R, D), lambda i: (i, 0)),
            pl.BlockSpec((D, D), lambda i: (0, 0)),
            pl.BlockSpec((D, D), lambda i: (0, 0)),
            pl.BlockSpec((1, D), lambda i: (0, 0)),
            pl.BlockSpec((D, 1), lambda i: (0, 0)),
            pl.BlockSpec((D, 1), lambda i: (0, 0)),
            pl.BlockSpec((1, 1), lambda i: (0, 0)),
        ],
        out_specs=pl.BlockSpec((_R, 1), lambda i: (i, 0)),
        out_shape=jax.ShapeDtypeStruct((NP, 1), jnp.float32),
    )(psum, pdeg3, h1, wl, wr, b, wa, wb, bro)


def kernel(x, edge_index, W1l, W1r, b1, W2l, W2r, b2, Wro, bro):
    xp = jnp.zeros((NP, D), jnp.float32).at[:N].set(x)
    src = edge_index[0]
    dst = edge_index[1]
    pad = jnp.arange(EP - E, dtype=jnp.int32)
    srcp = jnp.concatenate([src, pad % N])
    dstp = jnp.concatenate([dst, N + pad % (NP - N)])
    zrows = jnp.zeros((RPT, D), jnp.float32)
    zcol = jnp.zeros((RPT,), jnp.float32)
    ones = jnp.ones((CH,), jnp.float32)

    psum1, pdeg = _sc_agg_deg(xp, srcp, dstp, zrows, zcol, ones)
    pdeg3 = pdeg[..., None]
    h1 = _dense1(psum1, pdeg3, xp, W1l, W1r, b1.reshape(1, D))
    psum2 = _sc_agg(h1, srcp, dstp, zrows)
    out = _dense2(psum2, pdeg3, h1, W2l, W2r, b2.reshape(1, D),
                  Wro[:D], Wro[D:], bro.reshape(1, 1))
    return out[:N]


# speedup vs baseline: 1.2062x; 1.2062x over previous
import functools

import jax
import jax.numpy as jnp
from jax import lax
from jax.experimental import pallas as pl
from jax.experimental.pallas import tpu as pltpu
from jax.experimental.pallas import tpu_sc as plsc

N = 10000
E = 320000
D = 128
NC = 2
NS = 16
NW = NC * NS
NP = 10240
CH = 128
EPW = 10240
EP = NW * EPW
NCHUNK = EPW // CH
NBUF = 2
RPT = NP // NS

_mesh = plsc.VectorSubcoreMesh(core_axis_name="c", subcore_axis_name="s")


def _agg_body(with_deg, *refs):
    if with_deg:
        (table_hbm, src_hbm, dst_hbm, zrows_hbm, zcol_hbm, ones_hbm,
         psum_hbm, pdeg_hbm,
         acc_sh, deg_sh, sidx_v, didx_v, rows_v, ones_v,
         g0, g1, s0, s1) = refs
    else:
        (table_hbm, src_hbm, dst_hbm, zrows_hbm,
         psum_hbm,
         acc_sh, sidx_v, didx_v, rows_v,
         g0, g1, s0, s1) = refs
    gsems = (g0, g1)
    ssems = (s0, s1)

    cid = lax.axis_index("c")
    sid = lax.axis_index("s")
    wid = sid * NC + cid
    r0 = sid * RPT

    pltpu.sync_copy(zrows_hbm, acc_sh.at[pl.ds(r0, RPT)])
    if with_deg:
        pltpu.sync_copy(zcol_hbm, deg_sh.at[pl.ds(r0, RPT)])
        pltpu.sync_copy(ones_hbm, ones_v)
    plsc.subcore_barrier()

    ebase = wid * EPW

    def load_and_gather(c, b):
        base = pl.multiple_of(ebase + c * CH, 8)
        pltpu.sync_copy(src_hbm.at[pl.ds(base, CH)], sidx_v.at[b])
        pltpu.sync_copy(dst_hbm.at[pl.ds(base, CH)], didx_v.at[b])
        pltpu.async_copy(table_hbm.at[sidx_v.at[b]], rows_v.at[b], gsems[b])

    for b in range(NBUF):
        load_and_gather(b, b)

    def group(g, carry):
        for b in range(NBUF):
            pltpu.make_async_copy(table_hbm.at[sidx_v.at[b]],
                                  rows_v.at[b], gsems[b]).wait()
            pltpu.async_copy(rows_v.at[b], acc_sh.at[didx_v.at[b]],
                             ssems[b], add=True)
            if with_deg:
                pltpu.async_copy(ones_v, deg_sh.at[didx_v.at[b]],
                                 ssems[b], add=True)
        for b in range(NBUF):
            c2 = NBUF * g + b + NBUF
            pltpu.make_async_copy(rows_v.at[b], acc_sh.at[didx_v.at[b]],
                                  ssems[b]).wait()
            if with_deg:
                pltpu.make_async_copy(ones_v, deg_sh.at[didx_v.at[b]],
                                      ssems[b]).wait()

            @pl.when(c2 < NCHUNK)
            def _():
                load_and_gather(c2, b)
        return carry

    lax.fori_loop(0, NCHUNK // NBUF, group, 0)

    plsc.subcore_barrier()
    pltpu.sync_copy(acc_sh.at[pl.ds(r0, RPT)], psum_hbm.at[cid, pl.ds(r0, RPT)])
    if with_deg:
        pltpu.sync_copy(deg_sh.at[pl.ds(r0, RPT)],
                        pdeg_hbm.at[cid, pl.ds(r0, RPT)])


def _sc_agg_deg(table, src, dst, zrows, zcol, ones):
    f = pl.kernel(
        functools.partial(_agg_body, True),
        mesh=_mesh,
        out_type=[
            jax.ShapeDtypeStruct((NC, NP, D), jnp.float32),
            jax.ShapeDtypeStruct((NC, NP), jnp.float32),
        ],
        scratch_types=[
            pltpu.VMEM_SHARED((NP, D), jnp.float32),
            pltpu.VMEM_SHARED((NP,), jnp.float32),
            pltpu.VMEM((NBUF, CH), jnp.int32),
            pltpu.VMEM((NBUF, CH), jnp.int32),
            pltpu.VMEM((NBUF, CH, D), jnp.float32),
            pltpu.VMEM((CH,), jnp.float32),
        ] + [pltpu.SemaphoreType.DMA] * (2 * NBUF),
    )
    return f(table, src, dst, zrows, zcol, ones)


def _sc_agg(table, src, dst, zrows):
    f = pl.kernel(
        functools.partial(_agg_body, False),
        mesh=_mesh,
        out_type=jax.ShapeDtypeStruct((NC, NP, D), jnp.float32),
        scratch_types=[
            pltpu.VMEM_SHARED((NP, D), jnp.float32),
            pltpu.VMEM((NBUF, CH), jnp.int32),
            pltpu.VMEM((NBUF, CH), jnp.int32),
            pltpu.VMEM((NBUF, CH, D), jnp.float32),
        ] + [pltpu.SemaphoreType.DMA] * (2 * NBUF),
    )
    return f(table, src, dst, zrows)


_DOT = functools.partial(
    lax.dot_general,
    dimension_numbers=(((1,), (0,)), ((), ())),
    preferred_element_type=jnp.float32,
    precision=lax.Precision.HIGHEST,
)

_R = 1024


def _dense1_body(ps_ref, dg_ref, x_ref, wl_ref, wr_ref, b_ref, o_ref):
    s = ps_ref[0] + ps_ref[1]
    dg = dg_ref[0] + dg_ref[1]
    agg = s / jnp.maximum(dg, 1.0)
    h = _DOT(agg, wl_ref[...]) + b_ref[...] + _DOT(x_ref[...], wr_ref[...])
    o_ref[...] = jnp.maximum(h, 0.0)


def _dense1(psum, pdeg3, x, wl, wr, b):
    grid = (NP // _R,)
    return pl.pallas_call(
        _dense1_body,
        grid=grid,
        in_specs=[
            pl.BlockSpec((NC, _R, D), lambda i: (0, i, 0)),
            pl.BlockSpec((NC, _R, 1), lambda i: (0, i, 0)),
            pl.BlockSpec((_R, D), lambda i: (i, 0)),
            pl.BlockSpec((D, D), lambda i: (0, 0)),
            pl.BlockSpec((D, D), lambda i: (0, 0)),
            pl.BlockSpec((1, D), lambda i: (0, 0)),
        ],
        out_specs=pl.BlockSpec((_R, D), lambda i: (i, 0)),
        out_shape=jax.ShapeDtypeStruct((NP, D), jnp.float32),
    )(psum, pdeg3, x, wl, wr, b)


def _dense2_body(ps_ref, dg_ref, h1_ref, wl_ref, wr_ref, b_ref,
                 wa_ref, wb_ref, bro_ref, o_ref):
    s = ps_ref[0] + ps_ref[1]
    dg = dg_ref[0] + dg_ref[1]
    agg = s / jnp.maximum(dg, 1.0)
    h1 = h1_ref[...]
    h2 = jnp.maximum(_DOT(agg, wl_ref[...]) + b_ref[...]
                     + _DOT(h1, wr_ref[...]), 0.0)
    o_ref[...] = _DOT(h1, wa_ref[...]) + _DOT(h2, wb_ref[...]) + bro_ref[...]


def _dense2(psum, pdeg3, h1, wl, wr, b, wa, wb, bro):
    grid = (NP // _R,)
    return pl.pallas_call(
        _dense2_body,
        grid=grid,
        in_specs=[
            pl.BlockSpec((NC, _R, D), lambda i: (0, i, 0)),
            pl.BlockSpec((NC, _R, 1), lambda i: (0, i, 0)),
            pl.BlockSpec((_R, D), lambda i: (i, 0)),
            pl.BlockSpec((D, D), lambda i: (0, 0)),
            pl.BlockSpec((D, D), lambda i: (0, 0)),
            pl.BlockSpec((1, D), lambda i: (0, 0)),
            pl.BlockSpec((D, 1), lambda i: (0, 0)),
            pl.BlockSpec((D, 1), lambda i: (0, 0)),
            pl.BlockSpec((1, 1), lambda i: (0, 0)),
        ],
        out_specs=pl.BlockSpec((_R, 1), lambda i: (i, 0)),
        out_shape=jax.ShapeDtypeStruct((NP, 1), jnp.float32),
    )(psum, pdeg3, h1, wl, wr, b, wa, wb, bro)


def kernel(x, edge_index, W1l, W1r, b1, W2l, W2r, b2, Wro, bro):
    xp = jnp.zeros((NP, D), jnp.float32).at[:N].set(x)
    src = edge_index[0]
    dst = edge_index[1]
    pad = jnp.arange(EP - E, dtype=jnp.int32)
    srcp = jnp.concatenate([src, pad % N])
    dstp = jnp.concatenate([dst, N + pad % (NP - N)])
    zrows = jnp.zeros((RPT, D), jnp.float32)
    zcol = jnp.zeros((RPT,), jnp.float32)
    ones = jnp.ones((CH,), jnp.float32)

    psum1, pdeg = _sc_agg_deg(xp, srcp, dstp, zrows, zcol, ones)
    pdeg3 = pdeg[..., None]
    h1 = _dense1(psum1, pdeg3, xp, W1l, W1r, b1.reshape(1, D))
    psum2 = _sc_agg(h1, srcp, dstp, zrows)
    out = _dense2(psum2, pdeg3, h1, W2l, W2r, b2.reshape(1, D),
                  Wro[:D], Wro[D:], bro.reshape(1, 1))
    return out[:N]


# speedup vs baseline: 1.2285x; 1.0185x over previous
import functools

import jax
import jax.numpy as jnp
from jax import lax
from jax.experimental import pallas as pl
from jax.experimental.pallas import tpu as pltpu
from jax.experimental.pallas import tpu_sc as plsc

N = 10000
E = 320000
D = 128
NC = 2
NS = 16
NW = NC * NS
NP = 10240
CH = 128
EPW = 10240
EP = NW * EPW
NCHUNK = EPW // CH
NSLAB = 2
SLAB = NCHUNK // NSLAB
NBUF = 2
RPT = NP // NS

_mesh = plsc.VectorSubcoreMesh(core_axis_name="c", subcore_axis_name="s")


def _agg_body(with_deg, *refs):
    if with_deg:
        (table_hbm, src_hbm, dst_hbm, zrows_hbm, zcol_hbm, ones_hbm,
         psum_hbm, pdeg_hbm,
         acc_sh, deg_sh, sidx_v, didx_v, rows_v, ones_v,
         g0, g1, s0, s1) = refs
    else:
        (table_hbm, src_hbm, dst_hbm, zrows_hbm,
         psum_hbm,
         acc_sh, sidx_v, didx_v, rows_v,
         g0, g1, s0, s1) = refs
    gsems = (g0, g1)
    ssems = (s0, s1)

    cid = lax.axis_index("c")
    sid = lax.axis_index("s")
    wid = sid * NC + cid
    r0 = sid * RPT

    pltpu.sync_copy(zrows_hbm, acc_sh.at[pl.ds(r0, RPT)])
    if with_deg:
        pltpu.sync_copy(zcol_hbm, deg_sh.at[pl.ds(r0, RPT)])
        pltpu.sync_copy(ones_hbm, ones_v)
    plsc.subcore_barrier()

    crow = wid * NCHUNK

    def fire_gather(j, b):
        pltpu.async_copy(table_hbm.at[sidx_v.at[j]], rows_v.at[b], gsems[b])

    for s in range(NSLAB):
        pltpu.sync_copy(src_hbm.at[pl.ds(crow + s * SLAB, SLAB)], sidx_v)
        pltpu.sync_copy(dst_hbm.at[pl.ds(crow + s * SLAB, SLAB)], didx_v)
        for b in range(NBUF):
            fire_gather(b, b)

        def group(g, carry):
            for b in range(NBUF):
                j = NBUF * g + b
                pltpu.make_async_copy(table_hbm.at[sidx_v.at[j]],
                                      rows_v.at[b], gsems[b]).wait()
                pltpu.async_copy(rows_v.at[b], acc_sh.at[didx_v.at[j]],
                                 ssems[b], add=True)
                if with_deg:
                    pltpu.async_copy(ones_v, deg_sh.at[didx_v.at[j]],
                                     ssems[b], add=True)
            for b in range(NBUF):
                j = NBUF * g + b
                j2 = j + NBUF
                pltpu.make_async_copy(rows_v.at[b], acc_sh.at[didx_v.at[j]],
                                      ssems[b]).wait()
                if with_deg:
                    pltpu.make_async_copy(ones_v, deg_sh.at[didx_v.at[j]],
                                          ssems[b]).wait()

                @pl.when(j2 < SLAB)
                def _():
                    fire_gather(j2, b)
            return carry

        lax.fori_loop(0, SLAB // NBUF, group, 0)

    plsc.subcore_barrier()
    pltpu.sync_copy(acc_sh.at[pl.ds(r0, RPT)], psum_hbm.at[cid, pl.ds(r0, RPT)])
    if with_deg:
        pltpu.sync_copy(deg_sh.at[pl.ds(r0, RPT)],
                        pdeg_hbm.at[cid, pl.ds(r0, RPT)])


def _sc_agg_deg(table, src, dst, zrows, zcol, ones):
    f = pl.kernel(
        functools.partial(_agg_body, True),
        mesh=_mesh,
        out_type=[
            jax.ShapeDtypeStruct((NC, NP, D), jnp.float32),
            jax.ShapeDtypeStruct((NC, NP), jnp.float32),
        ],
        scratch_types=[
            pltpu.VMEM_SHARED((NP, D), jnp.float32),
            pltpu.VMEM_SHARED((NP,), jnp.float32),
            pltpu.VMEM((SLAB, CH), jnp.int32),
            pltpu.VMEM((SLAB, CH), jnp.int32),
            pltpu.VMEM((NBUF, CH, D), jnp.float32),
            pltpu.VMEM((CH,), jnp.float32),
        ] + [pltpu.SemaphoreType.DMA] * (2 * NBUF),
    )
    return f(table, src, dst, zrows, zcol, ones)


def _sc_agg(table, src, dst, zrows):
    f = pl.kernel(
        functools.partial(_agg_body, False),
        mesh=_mesh,
        out_type=jax.ShapeDtypeStruct((NC, NP, D), jnp.float32),
        scratch_types=[
            pltpu.VMEM_SHARED((NP, D), jnp.float32),
            pltpu.VMEM((SLAB, CH), jnp.int32),
            pltpu.VMEM((SLAB, CH), jnp.int32),
            pltpu.VMEM((NBUF, CH, D), jnp.float32),
        ] + [pltpu.SemaphoreType.DMA] * (2 * NBUF),
    )
    return f(table, src, dst, zrows)


_DOT = functools.partial(
    lax.dot_general,
    dimension_numbers=(((1,), (0,)), ((), ())),
    preferred_element_type=jnp.float32,
    precision=lax.Precision.HIGHEST,
)

_R = 1024


def _dense1_body(ps_ref, dg_ref, x_ref, wl_ref, wr_ref, b_ref, o_ref):
    s = ps_ref[0] + ps_ref[1]
    dg = dg_ref[0] + dg_ref[1]
    agg = s / jnp.maximum(dg, 1.0)
    h = _DOT(agg, wl_ref[...]) + b_ref[...] + _DOT(x_ref[...], wr_ref[...])
    o_ref[...] = jnp.maximum(h, 0.0)


def _dense1(psum, pdeg3, x, wl, wr, b):
    grid = (NP // _R,)
    return pl.pallas_call(
        _dense1_body,
        grid=grid,
        in_specs=[
            pl.BlockSpec((NC, _R, D), lambda i: (0, i, 0)),
            pl.BlockSpec((NC, _R, 1), lambda i: (0, i, 0)),
            pl.BlockSpec((_R, D), lambda i: (i, 0)),
            pl.BlockSpec((D, D), lambda i: (0, 0)),
            pl.BlockSpec((D, D), lambda i: (0, 0)),
            pl.BlockSpec((1, D), lambda i: (0, 0)),
        ],
        out_specs=pl.BlockSpec((_R, D), lambda i: (i, 0)),
        out_shape=jax.ShapeDtypeStruct((NP, D), jnp.float32),
    )(psum, pdeg3, x, wl, wr, b)


def _dense2_body(ps_ref, dg_ref, h1_ref, wl_ref, wr_ref, b_ref,
                 wa_ref, wb_ref, bro_ref, o_ref):
    s = ps_ref[0] + ps_ref[1]
    dg = dg_ref[0] + dg_ref[1]
    agg = s / jnp.maximum(dg, 1.0)
    h1 = h1_ref[...]
    h2 = jnp.maximum(_DOT(agg, wl_ref[...]) + b_ref[...]
                     + _DOT(h1, wr_ref[...]), 0.0)
    o_ref[...] = _DOT(h1, wa_ref[...]) + _DOT(h2, wb_ref[...]) + bro_ref[...]


def _dense2(psum, pdeg3, h1, wl, wr, b, wa, wb, bro):
    grid = (NP // _R,)
    return pl.pallas_call(
        _dense2_body,
        grid=grid,
        in_specs=[
            pl.BlockSpec((NC, _R, D), lambda i: (0, i, 0)),
            pl.BlockSpec((NC, _R, 1), lambda i: (0, i, 0)),
            pl.BlockSpec((_R, D), lambda i: (i, 0)),
            pl.BlockSpec((D, D), lambda i: (0, 0)),
            pl.BlockSpec((D, D), lambda i: (0, 0)),
            pl.BlockSpec((1, D), lambda i: (0, 0)),
            pl.BlockSpec((D, 1), lambda i: (0, 0)),
            pl.BlockSpec((D, 1), lambda i: (0, 0)),
            pl.BlockSpec((1, 1), lambda i: (0, 0)),
        ],
        out_specs=pl.BlockSpec((_R, 1), lambda i: (i, 0)),
        out_shape=jax.ShapeDtypeStruct((NP, 1), jnp.float32),
    )(psum, pdeg3, h1, wl, wr, b, wa, wb, bro)


def kernel(x, edge_index, W1l, W1r, b1, W2l, W2r, b2, Wro, bro):
    xp = jnp.zeros((NP, D), jnp.float32).at[:N].set(x)
    src = edge_index[0]
    dst = edge_index[1]
    pad = jnp.arange(EP - E, dtype=jnp.int32)
    srcp = jnp.concatenate([src, pad % N]).reshape(EP // CH, CH)
    dstp = jnp.concatenate([dst, N + pad % (NP - N)]).reshape(EP // CH, CH)
    zrows = jnp.zeros((RPT, D), jnp.float32)
    zcol = jnp.zeros((RPT,), jnp.float32)
    ones = jnp.ones((CH,), jnp.float32)

    psum1, pdeg = _sc_agg_deg(xp, srcp, dstp, zrows, zcol, ones)
    pdeg3 = pdeg[..., None]
    h1 = _dense1(psum1, pdeg3, xp, W1l, W1r, b1.reshape(1, D))
    psum2 = _sc_agg(h1, srcp, dstp, zrows)
    out = _dense2(psum2, pdeg3, h1, W2l, W2r, b2.reshape(1, D),
                  Wro[:D], Wro[D:], bro.reshape(1, 1))
    return out[:N]


# speedup vs baseline: 1.2427x; 1.0115x over previous
import functools

import jax
import jax.numpy as jnp
from jax import lax
from jax.experimental import pallas as pl
from jax.experimental.pallas import tpu as pltpu
from jax.experimental.pallas import tpu_sc as plsc

N = 10000
E = 320000
D = 128
NC = 2
NS = 16
NW = NC * NS
NP = 10240
CH = 128
EPW = 10240
EP = NW * EPW
NCHUNK = EPW // CH
NSLAB = 2
SLAB = NCHUNK // NSLAB
NBUF = 2
RPT = NP // NS

_mesh = plsc.VectorSubcoreMesh(core_axis_name="c", subcore_axis_name="s")


def _agg_body(with_deg, *refs):
    if with_deg:
        (table_hbm, src_hbm, dst_hbm, zrows_hbm, zcol_hbm, ones_hbm,
         psum_hbm, pdeg_hbm,
         acc_sh, deg_sh, sidx_v, didx_v, rows_v, ones_v,
         g0, g1, s0, s1) = refs
    else:
        (table_hbm, src_hbm, dst_hbm, zrows_hbm,
         psum_hbm,
         acc_sh, sidx_v, didx_v, rows_v,
         g0, g1, s0, s1) = refs
    gsems = (g0, g1)
    ssems = (s0, s1)

    cid = lax.axis_index("c")
    sid = lax.axis_index("s")
    wid = sid * NC + cid
    r0 = sid * RPT

    pltpu.sync_copy(zrows_hbm, acc_sh.at[pl.ds(r0, RPT)])
    if with_deg:
        pltpu.sync_copy(zcol_hbm, deg_sh.at[pl.ds(r0, RPT)])
        pltpu.sync_copy(ones_hbm, ones_v)
    plsc.subcore_barrier()

    crow = wid * NCHUNK

    def fire_gather(j, b):
        pltpu.async_copy(table_hbm.at[sidx_v.at[j]], rows_v.at[b], gsems[b])

    for s in range(NSLAB):
        pltpu.sync_copy(src_hbm.at[pl.ds(crow + s * SLAB, SLAB)], sidx_v)
        pltpu.sync_copy(dst_hbm.at[pl.ds(crow + s * SLAB, SLAB)], didx_v)
        for b in range(NBUF):
            fire_gather(b, b)

        def group(g, carry):
            for b in range(NBUF):
                j = NBUF * g + b
                pltpu.make_async_copy(table_hbm.at[sidx_v.at[j]],
                                      rows_v.at[b], gsems[b]).wait()
                pltpu.async_copy(rows_v.at[b], acc_sh.at[didx_v.at[j]],
                                 ssems[b], add=True)
                if with_deg:
                    pltpu.async_copy(ones_v, deg_sh.at[didx_v.at[j]],
                                     ssems[b], add=True)
            for b in range(NBUF):
                j = NBUF * g + b
                j2 = j + NBUF
                pltpu.make_async_copy(rows_v.at[b], acc_sh.at[didx_v.at[j]],
                                      ssems[b]).wait()
                if with_deg:
                    pltpu.make_async_copy(ones_v, deg_sh.at[didx_v.at[j]],
                                          ssems[b]).wait()

                @pl.when(j2 < SLAB)
                def _():
                    fire_gather(j2, b)
            return carry

        lax.fori_loop(0, SLAB // NBUF, group, 0)

    plsc.subcore_barrier()
    pltpu.sync_copy(acc_sh.at[pl.ds(r0, RPT)], psum_hbm.at[cid, pl.ds(r0, RPT)])
    if with_deg:
        pltpu.sync_copy(deg_sh.at[pl.ds(r0, RPT)],
                        pdeg_hbm.at[cid, pl.ds(r0, RPT)])


def _sc_agg_deg(table, src, dst, zrows, zcol, ones):
    f = pl.kernel(
        functools.partial(_agg_body, True),
        mesh=_mesh,
        out_type=[
            jax.ShapeDtypeStruct((NC, NP, D), jnp.float32),
            jax.ShapeDtypeStruct((NC, NP), jnp.float32),
        ],
        scratch_types=[
            pltpu.VMEM_SHARED((NP, D), jnp.float32),
            pltpu.VMEM_SHARED((NP,), jnp.float32),
            pltpu.VMEM((SLAB, CH), jnp.int32),
            pltpu.VMEM((SLAB, CH), jnp.int32),
            pltpu.VMEM((NBUF, CH, D), jnp.float32),
            pltpu.VMEM((CH,), jnp.float32),
        ] + [pltpu.SemaphoreType.DMA] * (2 * NBUF),
    )
    return f(table, src, dst, zrows, zcol, ones)


def _sc_agg(table, src, dst, zrows):
    f = pl.kernel(
        functools.partial(_agg_body, False),
        mesh=_mesh,
        out_type=jax.ShapeDtypeStruct((NC, NP, D), jnp.float32),
        scratch_types=[
            pltpu.VMEM_SHARED((NP, D), jnp.float32),
            pltpu.VMEM((SLAB, CH), jnp.int32),
            pltpu.VMEM((SLAB, CH), jnp.int32),
            pltpu.VMEM((NBUF, CH, D), jnp.float32),
        ] + [pltpu.SemaphoreType.DMA] * (2 * NBUF),
    )
    return f(table, src, dst, zrows)


_DOT = functools.partial(
    lax.dot_general,
    dimension_numbers=(((1,), (0,)), ((), ())),
    preferred_element_type=jnp.float32,
    precision=lax.Precision.HIGHEST,
)

_R = 1000


def _root1_body(x_ref, wr_ref, b_ref, o_ref):
    o_ref[...] = _DOT(x_ref[...], wr_ref[...]) + b_ref[...]


def _root1(x, wr, b):
    return pl.pallas_call(
        _root1_body,
        grid=(N // _R,),
        in_specs=[
            pl.BlockSpec((_R, D), lambda i: (i, 0)),
            pl.BlockSpec((D, D), lambda i: (0, 0)),
            pl.BlockSpec((1, D), lambda i: (0, 0)),
        ],
        out_specs=pl.BlockSpec((_R, D), lambda i: (i, 0)),
        out_shape=jax.ShapeDtypeStruct((N, D), jnp.float32),
    )(x, wr, b)


def _dense1_body(ps_ref, dg_ref, r_ref, wl_ref, o_ref):
    s = ps_ref[0] + ps_ref[1]
    dg = dg_ref[0] + dg_ref[1]
    agg = s / jnp.maximum(dg, 1.0)
    o_ref[...] = jnp.maximum(_DOT(agg, wl_ref[...]) + r_ref[...], 0.0)


def _dense1(psum, pdeg3, r1, wl):
    return pl.pallas_call(
        _dense1_body,
        grid=(N // _R,),
        in_specs=[
            pl.BlockSpec((NC, _R, D), lambda i: (0, i, 0)),
            pl.BlockSpec((NC, _R, 1), lambda i: (0, i, 0)),
            pl.BlockSpec((_R, D), lambda i: (i, 0)),
            pl.BlockSpec((D, D), lambda i: (0, 0)),
        ],
        out_specs=pl.BlockSpec((_R, D), lambda i: (i, 0)),
        out_shape=jax.ShapeDtypeStruct((N, D), jnp.float32),
    )(psum, pdeg3, r1, wl)


def _root2_body(h1_ref, wr_ref, b_ref, wa_ref, r_ref, o1_ref):
    h1 = h1_ref[...]
    r_ref[...] = _DOT(h1, wr_ref[...]) + b_ref[...]
    o1_ref[...] = _DOT(h1, wa_ref[...])


def _root2(h1, wr, b, wa):
    return pl.pallas_call(
        _root2_body,
        grid=(N // _R,),
        in_specs=[
            pl.BlockSpec((_R, D), lambda i: (i, 0)),
            pl.BlockSpec((D, D), lambda i: (0, 0)),
            pl.BlockSpec((1, D), lambda i: (0, 0)),
            pl.BlockSpec((D, 1), lambda i: (0, 0)),
        ],
        out_specs=[
            pl.BlockSpec((_R, D), lambda i: (i, 0)),
            pl.BlockSpec((_R, 1), lambda i: (i, 0)),
        ],
        out_shape=[
            jax.ShapeDtypeStruct((N, D), jnp.float32),
            jax.ShapeDtypeStruct((N, 1), jnp.float32),
        ],
    )(h1, wr, b, wa)


def _dense2_body(ps_ref, dg_ref, r_ref, o1_ref, wl_ref, wb_ref, bro_ref,
                 o_ref):
    s = ps_ref[0] + ps_ref[1]
    dg = dg_ref[0] + dg_ref[1]
    agg = s / jnp.maximum(dg, 1.0)
    h2 = jnp.maximum(_DOT(agg, wl_ref[...]) + r_ref[...], 0.0)
    o_ref[...] = o1_ref[...] + _DOT(h2, wb_ref[...]) + bro_ref[...]


def _dense2(psum, pdeg3, r2, o1, wl, wb, bro):
    return pl.pallas_call(
        _dense2_body,
        grid=(N // _R,),
        in_specs=[
            pl.BlockSpec((NC, _R, D), lambda i: (0, i, 0)),
            pl.BlockSpec((NC, _R, 1), lambda i: (0, i, 0)),
            pl.BlockSpec((_R, D), lambda i: (i, 0)),
            pl.BlockSpec((_R, 1), lambda i: (i, 0)),
            pl.BlockSpec((D, D), lambda i: (0, 0)),
            pl.BlockSpec((D, 1), lambda i: (0, 0)),
            pl.BlockSpec((1, 1), lambda i: (0, 0)),
        ],
        out_specs=pl.BlockSpec((_R, 1), lambda i: (i, 0)),
        out_shape=jax.ShapeDtypeStruct((N, 1), jnp.float32),
    )(psum, pdeg3, r2, o1, wl, wb, bro)


def kernel(x, edge_index, W1l, W1r, b1, W2l, W2r, b2, Wro, bro):
    src = edge_index[0]
    dst = edge_index[1]
    pad = jnp.arange(EP - E, dtype=jnp.int32)
    srcp = jnp.concatenate([src, pad % N]).reshape(EP // CH, CH)
    dstp = jnp.concatenate([dst, N + pad % (NP - N)]).reshape(EP // CH, CH)
    zrows = jnp.zeros((RPT, D), jnp.float32)
    zcol = jnp.zeros((RPT,), jnp.float32)
    ones = jnp.ones((CH,), jnp.float32)

    r1 = _root1(x, W1r, b1.reshape(1, D))
    psum1, pdeg = _sc_agg_deg(x, srcp, dstp, zrows, zcol, ones)
    pdeg3 = pdeg[..., None]
    h1 = _dense1(psum1, pdeg3, r1, W1l)
    r2, o1 = _root2(h1, W2r, b2.reshape(1, D), Wro[:D])
    psum2 = _sc_agg(h1, srcp, dstp, zrows)
    out = _dense2(psum2, pdeg3, r2, o1, W2l, Wro[D:], bro.reshape(1, 1))
    return out
